# Initial kernel scaffold; baseline (speedup 1.0000x reference)
import functools

import jax
import jax.numpy as jnp
from jax import lax
from jax.experimental import pallas as pl
from jax.experimental.pallas import tpu as pltpu
from jax.experimental.pallas import tpu_sc as plsc

N = 10000
E = 320000
NC, NS, L = 2, 16, 16
NW = NC * NS
EPW = E // NW
K = 80
NCHUNK = EPW // K
RPS = N // NS

_mesh = plsc.VectorSubcoreMesh(core_axis_name="c", subcore_axis_name="s",
                               num_cores=NC, num_subcores=NS)


def _fill(ref, nrows, ncols, value):
    v = jnp.full((L,), value, jnp.float32)

    @pl.loop(0, nrows)
    def _(r):
        @pl.loop(0, ncols, step=L)
        def _(c):
            ref.at[r, pl.ds(c, L)][...] = v


def _sc_degree(dst):
    @functools.partial(
        pl.kernel,
        out_type=jax.ShapeDtypeStruct((NC, N, L), jnp.float32),
        mesh=_mesh,
        scratch_types=[
            pltpu.VMEM((K,), jnp.int32),
            pltpu.VMEM((K, L), jnp.float32),
            pltpu.VMEM((RPS, L), jnp.float32),
            pltpu.VMEM_SHARED((N, L), jnp.float32),
        ],
    )
    def deg_kernel(dst_hbm, out_hbm, dstv, onesv, zerov, acc):
        cid = lax.axis_index("c")
        sid = lax.axis_index("s")
        _fill(onesv, K, L, 1.0)
        _fill(zerov, RPS, L, 0.0)
        base_row = sid * RPS
        pltpu.sync_copy(zerov, acc.at[pl.ds(base_row, RPS)])
        plsc.subcore_barrier()
        base_edge = (cid * NS + sid) * EPW

        @pl.loop(0, NCHUNK)
        def _(i):
            pltpu.sync_copy(dst_hbm.at[pl.ds(base_edge + i * K, K)], dstv)
            pltpu.sync_copy(onesv, acc.at[dstv], add=True)

        plsc.subcore_barrier()
        pltpu.sync_copy(acc.at[pl.ds(base_row, RPS)],
                        out_hbm.at[cid, pl.ds(base_row, RPS)])

    return deg_kernel(dst)


def _sc_aggregate(g, src, dst, f):
    @functools.partial(
        pl.kernel,
        out_type=jax.ShapeDtypeStruct((NC, N, f), jnp.float32),
        mesh=_mesh,
        scratch_types=[
            pltpu.VMEM((K,), jnp.int32),
            pltpu.VMEM((K,), jnp.int32),
            pltpu.VMEM((K, f), jnp.float32),
            pltpu.VMEM((RPS, f), jnp.float32),
            pltpu.VMEM_SHARED((N, f), jnp.float32),
            pltpu.SemaphoreType.DMA,
        ],
    )
    def agg_kernel(g_hbm, src_hbm, dst_hbm, out_hbm, srcv, dstv, rowsv, zerov,
                   acc, sem):
        cid = lax.axis_index("c")
        sid = lax.axis_index("s")
        _fill(zerov, RPS, f, 0.0)
        base_row = sid * RPS
        pltpu.sync_copy(zerov, acc.at[pl.ds(base_row, RPS)])
        plsc.subcore_barrier()
        base_edge = (cid * NS + sid) * EPW

        @pl.loop(0, NCHUNK)
        def _(i):
            off = base_edge + i * K
            pltpu.sync_copy(src_hbm.at[pl.ds(off, K)], srcv)
            pltpu.sync_copy(dst_hbm.at[pl.ds(off, K)], dstv)
            pltpu.async_copy(g_hbm.at[srcv], rowsv, sem).wait()
            pltpu.sync_copy(rowsv, acc.at[dstv], add=True)

        plsc.subcore_barrier()
        pltpu.sync_copy(acc.at[pl.ds(base_row, RPS)],
                        out_hbm.at[cid, pl.ds(base_row, RPS)])

    return agg_kernel(g, src, dst)


def _dinv(d_ref):
    return lax.rsqrt(1.0 + d_ref[0, :, 0:1] + d_ref[1, :, 0:1])


def _dot(a, b):
    return lax.dot_general(a, b, (((1,), (0,)), ((), ())),
                           preferred_element_type=jnp.float32,
                           precision=lax.Precision.HIGHEST)


def _tc_matmul(x, w):
    def body(x_ref, w_ref, o_ref):
        o_ref[...] = _dot(x_ref[...], w_ref[...])

    return pl.pallas_call(
        body,
        out_shape=jax.ShapeDtypeStruct((x.shape[0], w.shape[1]), jnp.float32),
    )(x, w)


def _tc_scale(degp, h):
    def body(d_ref, h_ref, o_ref):
        o_ref[...] = _dinv(d_ref) * h_ref[...]

    return pl.pallas_call(
        body, out_shape=jax.ShapeDtypeStruct(h.shape, jnp.float32)
    )(degp, h)


def _tc_combine(degp, p, g, b, w):
    def body(d_ref, p_ref, g_ref, b_ref, w_ref, o_ref):
        dinv = _dinv(d_ref)
        t = dinv * (p_ref[0] + p_ref[1] + g_ref[...]) + b_ref[...]
        t = jnp.maximum(t, 0.0)
        o_ref[...] = dinv * _dot(t, w_ref[...])

    return pl.pallas_call(
        body,
        out_shape=jax.ShapeDtypeStruct((g.shape[0], w.shape[1]), jnp.float32),
    )(degp, p, g, b, w)


def _tc_final(degp, p, g, b):
    def body(d_ref, p_ref, g_ref, b_ref, o_ref):
        o_ref[...] = _dinv(d_ref) * (p_ref[0] + p_ref[1] + g_ref[...]) + b_ref[...]

    return pl.pallas_call(
        body, out_shape=jax.ShapeDtypeStruct(g.shape, jnp.float32)
    )(degp, p, g, b)


def kernel(x, edge_index, W1, b1, W2, b2, W3, b3):
    src = edge_index[0]
    dst = edge_index[1]

    degp = _sc_degree(dst)
    h1 = _tc_matmul(x, W1)
    g1 = _tc_scale(degp, h1)
    p1 = _sc_aggregate(g1, src, dst, 64)

    g2 = _tc_combine(degp, p1, g1, b1.reshape(1, -1), W2)
    p2 = _sc_aggregate(g2, src, dst, 64)

    W3p = jnp.pad(W3, ((0, 0), (0, 8)))
    b3p = jnp.pad(b3, (0, 8))
    g3 = _tc_combine(degp, p2, g2, b2.reshape(1, -1), W3p)
    p3 = _sc_aggregate(g3, src, dst, 48)

    out = _tc_final(degp, p3, g3, b3p.reshape(1, -1))
    return out[:, :40]


# speedup vs baseline: 12.6881x; 12.6881x over previous
import functools

import jax
import jax.numpy as jnp
from jax import lax
from jax.experimental import pallas as pl
from jax.experimental.pallas import tpu as pltpu
from jax.experimental.pallas import tpu_sc as plsc

N = 10000
E = 320000
NC, NS, L = 2, 16, 16
NW = NC * NS
EPW = E // NW
K = 80
NCHUNK = EPW // K
NPAD = 10240
RPS = NPAD // NS

_mesh = plsc.VectorSubcoreMesh(core_axis_name="c", subcore_axis_name="s",
                               num_cores=NC, num_subcores=NS)
_sc_params = pltpu.CompilerParams(use_tc_tiling_on_sc=False)


def _fill(ref, nrows, ncols, value):
    v = jnp.full((L,), value, jnp.float32)

    @pl.loop(0, nrows)
    def _(r):
        @pl.loop(0, ncols, step=L)
        def _(c):
            ref.at[r, pl.ds(c, L)][...] = v


def _sc_degree(dst):
    @functools.partial(
        pl.kernel,
        out_type=jax.ShapeDtypeStruct((NC, NPAD, L), jnp.float32),
        mesh=_mesh,
        compiler_params=_sc_params,
        scratch_types=[
            pltpu.VMEM((K,), jnp.int32),
            pltpu.VMEM((K, L), jnp.float32),
            pltpu.VMEM((RPS, L), jnp.float32),
            pltpu.VMEM_SHARED((NPAD, L), jnp.float32),
        ],
    )
    def deg_kernel(dst_hbm, out_hbm, dstv, onesv, zerov, acc):
        cid = lax.axis_index("c")
        sid = lax.axis_index("s")
        _fill(onesv, K, L, 1.0)
        _fill(zerov, RPS, L, 0.0)
        base_row = sid * RPS
        pltpu.sync_copy(zerov, acc.at[pl.ds(base_row, RPS)])
        plsc.subcore_barrier()
        base_edge = (cid * NS + sid) * EPW

        @pl.loop(0, NCHUNK)
        def _(i):
            pltpu.sync_copy(dst_hbm.at[pl.ds(base_edge + i * K, K)], dstv)
            pltpu.sync_copy(onesv, acc.at[dstv], add=True)

        plsc.subcore_barrier()
        pltpu.sync_copy(acc.at[pl.ds(base_row, RPS)],
                        out_hbm.at[cid, pl.ds(base_row, RPS)])

    return deg_kernel(dst)


def _sc_aggregate(g, src, dst, f):
    @functools.partial(
        pl.kernel,
        out_type=jax.ShapeDtypeStruct((NC, NPAD, f), jnp.float32),
        mesh=_mesh,
        compiler_params=_sc_params,
        scratch_types=[
            pltpu.VMEM((K,), jnp.int32),
            pltpu.VMEM((K,), jnp.int32),
            pltpu.VMEM((K, f), jnp.float32),
            pltpu.VMEM((RPS, f), jnp.float32),
            pltpu.VMEM_SHARED((NPAD, f), jnp.float32),
            pltpu.SemaphoreType.DMA,
        ],
    )
    def agg_kernel(g_hbm, src_hbm, dst_hbm, out_hbm, srcv, dstv, rowsv, zerov,
                   acc, sem):
        cid = lax.axis_index("c")
        sid = lax.axis_index("s")
        _fill(zerov, RPS, f, 0.0)
        base_row = sid * RPS
        pltpu.sync_copy(zerov, acc.at[pl.ds(base_row, RPS)])
        plsc.subcore_barrier()
        base_edge = (cid * NS + sid) * EPW

        @pl.loop(0, NCHUNK)
        def _(i):
            off = base_edge + i * K
            pltpu.sync_copy(src_hbm.at[pl.ds(off, K)], srcv)
            pltpu.sync_copy(dst_hbm.at[pl.ds(off, K)], dstv)
            pltpu.async_copy(g_hbm.at[srcv], rowsv, sem).wait()
            pltpu.sync_copy(rowsv, acc.at[dstv], add=True)

        plsc.subcore_barrier()
        pltpu.sync_copy(acc.at[pl.ds(base_row, RPS)],
                        out_hbm.at[cid, pl.ds(base_row, RPS)])

    return agg_kernel(g, src, dst)


def _dinv(d_ref):
    return lax.rsqrt(1.0 + d_ref[0, 0:N, 0:1] + d_ref[1, 0:N, 0:1])


def _dot(a, b):
    return lax.dot_general(a, b, (((1,), (0,)), ((), ())),
                           preferred_element_type=jnp.float32,
                           precision=lax.Precision.HIGHEST)


def _tc_matmul(x, w):
    def body(x_ref, w_ref, o_ref):
        o_ref[...] = _dot(x_ref[...], w_ref[...])

    return pl.pallas_call(
        body,
        out_shape=jax.ShapeDtypeStruct((x.shape[0], w.shape[1]), jnp.float32),
    )(x, w)


def _tc_scale(degp, h):
    def body(d_ref, h_ref, o_ref):
        o_ref[...] = _dinv(d_ref) * h_ref[...]

    return pl.pallas_call(
        body, out_shape=jax.ShapeDtypeStruct(h.shape, jnp.float32)
    )(degp, h)


def _tc_combine(degp, p, g, b, w):
    def body(d_ref, p_ref, g_ref, b_ref, w_ref, o_ref):
        dinv = _dinv(d_ref)
        t = dinv * (p_ref[0, 0:N] + p_ref[1, 0:N] + g_ref[...]) + b_ref[...]
        t = jnp.maximum(t, 0.0)
        o_ref[...] = dinv * _dot(t, w_ref[...])

    return pl.pallas_call(
        body,
        out_shape=jax.ShapeDtypeStruct((g.shape[0], w.shape[1]), jnp.float32),
    )(degp, p, g, b, w)


def _tc_final(degp, p, g, b):
    def body(d_ref, p_ref, g_ref, b_ref, o_ref):
        o_ref[...] = _dinv(d_ref) * (p_ref[0, 0:N] + p_ref[1, 0:N] + g_ref[...]) + b_ref[...]

    return pl.pallas_call(
        body, out_shape=jax.ShapeDtypeStruct(g.shape, jnp.float32)
    )(degp, p, g, b)


def kernel(x, edge_index, W1, b1, W2, b2, W3, b3):
    src = edge_index[0]
    dst = edge_index[1]

    degp = _sc_degree(dst)
    h1 = _tc_matmul(x, W1)
    g1 = _tc_scale(degp, h1)
    p1 = _sc_aggregate(g1, src, dst, 64)

    g2 = _tc_combine(degp, p1, g1, b1.reshape(1, -1), W2)
    p2 = _sc_aggregate(g2, src, dst, 64)

    W3p = jnp.pad(W3, ((0, 0), (0, 8)))
    b3p = jnp.pad(b3, (0, 8))
    g3 = _tc_combine(degp, p2, g2, b2.reshape(1, -1), W3p)
    p3 = _sc_aggregate(g3, src, dst, 48)

    out = _tc_final(degp, p3, g3, b3p.reshape(1, -1))
    return out[:, :40]
